# baseline (speedup 1.0000x reference)
import jax
import jax.numpy as jnp
from jax import lax
from jax.experimental import pallas as pl

N = 16384
C = 1000
NB = 16
W = N // NB
BR = 128
GRID = N // BR
S = 128
SEARCH_ITERS = 31


def _phase_a_kernel(x_ref, lbl_ref, conf_ref, acc_ref):
    x = x_ref[...]
    conf_ref[...] = jnp.max(x, axis=1, keepdims=True)
    pidx = jnp.argmax(x, axis=1, keepdims=True)
    acc_ref[...] = (pidx == lbl_ref[...]).astype(jnp.float32)


def _phase_b_kernel(conf_ref, acc_ref, ece_ref, ys_ref):
    conf = conf_ref[...]
    acc = acc_ref[...]
    u = lax.bitcast_convert_type(conf, jnp.int32)

    sub16 = lax.broadcasted_iota(jnp.int32, (NB, 1), 0)
    rank_tgt = jnp.where(sub16 < NB - 1, (sub16 + 1) * W, N + 8)
    rank_tgt_f = rank_tgt.astype(jnp.float32) + 1.0

    def count_le(bound_col):
        accm = jnp.zeros((NB, S), jnp.float32)
        for r in range(S):
            urow = u[r:r + 1, :]
            accm = accm + jnp.where(urow <= bound_col, 1.0, 0.0)
        return jnp.sum(accm, axis=1, keepdims=True)

    def search_body(_, carry):
        lo, hi = carry
        mid = lo + lax.shift_right_arithmetic(hi - lo, jnp.int32(1))
        pred = count_le(mid) >= rank_tgt_f
        return (jnp.where(pred, lo, mid + 1), jnp.where(pred, mid, hi))

    lo0 = jnp.zeros((NB, 1), jnp.int32)
    hi0 = jnp.full((NB, 1), 0x40000000, jnp.int32)
    v_col, _ = lax.fori_loop(0, SEARCH_ITERS, search_body, (lo0, hi0))

    accm = jnp.zeros((NB, S), jnp.float32)
    for r in range(S):
        urow = u[r:r + 1, :]
        accm = accm + jnp.where(urow < v_col, 1.0, 0.0)
    n_low_col = rank_tgt.astype(jnp.float32) - jnp.sum(accm, axis=1,
                                                       keepdims=True)

    r_iota = lax.broadcasted_iota(jnp.int32, (S, S), 0)
    c_iota = lax.broadcasted_iota(jnp.int32, (S, S), 1)
    l_strict = (c_iota < r_iota).astype(jnp.float32)
    u_strict = (r_iota < c_iota).astype(jnp.float32)
    ones_mat = jnp.ones((S, S), jnp.float32)
    binf = jnp.zeros((S, S), jnp.float32)
    for b in range(NB - 1):
        vb = v_col[b:b + 1, 0:1]
        nl = n_low_col[b:b + 1, 0:1]
        mb = (u == vb).astype(jnp.float32)
        t1 = jnp.dot(mb, u_strict, preferred_element_type=jnp.float32)
        rowtot = jnp.dot(mb, ones_mat, preferred_element_type=jnp.float32)
        t2 = jnp.dot(l_strict, rowtot, preferred_element_type=jnp.float32)
        tier = t1 + t2
        binf = (binf + jnp.where(u > vb, 1.0, 0.0)
                + jnp.where((u == vb) & (tier >= nl), 1.0, 0.0))

    lane16 = lax.broadcasted_iota(jnp.int32, (1, NB), 1).astype(jnp.float32)
    scc = jnp.zeros((S, NB), jnp.float32)
    sac = jnp.zeros((S, NB), jnp.float32)
    for c in range(S):
        bcol = binf[:, c:c + 1]
        onehot = jnp.where(bcol == lane16, 1.0, 0.0)
        scc = scc + conf[:, c:c + 1] * onehot
        sac = sac + acc[:, c:c + 1] * onehot
    avg_conf = jnp.sum(scc, axis=0, keepdims=True) * (1.0 / W)
    ys_row = jnp.sum(sac, axis=0, keepdims=True) * (1.0 / W)
    ece = jnp.sum(jnp.abs(avg_conf - ys_row)) * (float(W) / float(N))
    ece_ref[...] = jnp.broadcast_to(ece, (1, 1))
    ys_ref[...] = ys_row


@jax.jit
def kernel(softmaxes, labels):
    lbl2 = labels.astype(jnp.int32).reshape(N, 1)
    conf, accv = pl.pallas_call(
        _phase_a_kernel,
        grid=(GRID,),
        in_specs=[pl.BlockSpec((BR, C), lambda i: (i, 0)),
                  pl.BlockSpec((BR, 1), lambda i: (i, 0))],
        out_specs=[pl.BlockSpec((BR, 1), lambda i: (i, 0)),
                   pl.BlockSpec((BR, 1), lambda i: (i, 0))],
        out_shape=[jax.ShapeDtypeStruct((N, 1), jnp.float32),
                   jax.ShapeDtypeStruct((N, 1), jnp.float32)],
    )(softmaxes, lbl2)

    ece, ys = pl.pallas_call(
        _phase_b_kernel,
        in_specs=[pl.BlockSpec((S, S), lambda: (0, 0)),
                  pl.BlockSpec((S, S), lambda: (0, 0))],
        out_specs=[pl.BlockSpec((1, 1), lambda: (0, 0)),
                   pl.BlockSpec((1, NB), lambda: (0, 0))],
        out_shape=[jax.ShapeDtypeStruct((1, 1), jnp.float32),
                   jax.ShapeDtypeStruct((1, NB), jnp.float32)],
    )(conf.reshape(S, S), accv.reshape(S, S))
    return (ece.reshape(1), ys.reshape(NB))


# speedup vs baseline: 1.3958x; 1.3958x over previous
import jax
import jax.numpy as jnp
from jax import lax
from jax.experimental import pallas as pl

N = 16384
C = 1000
NB = 16
W = N // NB
BR = 256
GRID = N // BR
S = 128
SEARCH_ITERS = 31


def _phase_a_kernel(x_ref, lbl_ref, conf_ref, acc_ref):
    x = x_ref[...]
    m = jnp.max(x, axis=1, keepdims=True)
    col = lax.broadcasted_iota(jnp.int32, x.shape, 1)
    big = jnp.int32(2 ** 30)
    pidx = jnp.min(jnp.where(x == m, col, big), axis=1, keepdims=True)
    conf_ref[...] = m
    acc_ref[...] = (pidx == lbl_ref[...]).astype(jnp.float32)


def _sum11(x):
    return jnp.sum(jnp.sum(x, axis=1, keepdims=True), axis=0, keepdims=True)


def _phase_b_kernel(conf_ref, acc_ref, ece_ref, ys_ref):
    conf = conf_ref[...]
    acc = acc_ref[...]
    u = lax.bitcast_convert_type(conf, jnp.int32)

    nb1 = NB - 1
    ranks_f = [jnp.full((1, 1), float((b + 1) * W), jnp.float32)
               for b in range(nb1)]

    def search_body(_, carry):
        los, his = carry
        nlos, nhis = [], []
        for b in range(nb1):
            lo, hi = los[b], his[b]
            mid = lo + lax.shift_right_arithmetic(hi - lo, jnp.int32(1))
            cnt = _sum11(jnp.where(u <= mid, 1.0, 0.0))
            pred = cnt >= ranks_f[b] + 1.0
            nlos.append(jnp.where(pred, lo, mid + 1))
            nhis.append(jnp.where(pred, mid, hi))
        return tuple(nlos), tuple(nhis)

    init = (tuple(jnp.zeros((1, 1), jnp.int32) for _ in range(nb1)),
            tuple(jnp.full((1, 1), 0x40000000, jnp.int32)
                  for _ in range(nb1)))
    vs, _ = lax.fori_loop(0, SEARCH_ITERS, search_body, init)

    r_iota = lax.broadcasted_iota(jnp.int32, (S, S), 0)
    c_iota = lax.broadcasted_iota(jnp.int32, (S, S), 1)
    l_strict = (c_iota < r_iota).astype(jnp.float32)
    u_strict = (r_iota < c_iota).astype(jnp.float32)
    ones_mat = jnp.ones((S, S), jnp.float32)
    binf = jnp.zeros((S, S), jnp.float32)
    for b in range(nb1):
        vb = vs[b]
        eq = (u == vb)
        nl = ranks_f[b] - _sum11(jnp.where(u < vb, 1.0, 0.0))
        mb = eq.astype(jnp.float32)
        t1 = jnp.dot(mb, u_strict, preferred_element_type=jnp.float32)
        rowtot = jnp.dot(mb, ones_mat, preferred_element_type=jnp.float32)
        t2 = jnp.dot(l_strict, rowtot, preferred_element_type=jnp.float32)
        tier = t1 + t2
        binf = (binf + jnp.where(u > vb, 1.0, 0.0)
                + jnp.where(eq & (tier >= nl), 1.0, 0.0))

    avgs, yss = [], []
    for k in range(NB):
        mk = (binf == float(k)).astype(jnp.float32)
        avgs.append(_sum11(conf * mk) * (1.0 / W))
        yss.append(_sum11(acc * mk) * (1.0 / W))
    avg_conf = jnp.concatenate(avgs, axis=1)
    ys_row = jnp.concatenate(yss, axis=1)
    ece = jnp.sum(jnp.abs(avg_conf - ys_row), axis=1,
                  keepdims=True) * (float(W) / float(N))
    ece_ref[...] = ece
    ys_ref[...] = ys_row


@jax.jit
def kernel(softmaxes, labels):
    lbl2 = labels.astype(jnp.int32).reshape(N, 1)
    conf, accv = pl.pallas_call(
        _phase_a_kernel,
        grid=(GRID,),
        in_specs=[pl.BlockSpec((BR, C), lambda i: (i, 0)),
                  pl.BlockSpec((BR, 1), lambda i: (i, 0))],
        out_specs=[pl.BlockSpec((BR, 1), lambda i: (i, 0)),
                   pl.BlockSpec((BR, 1), lambda i: (i, 0))],
        out_shape=[jax.ShapeDtypeStruct((N, 1), jnp.float32),
                   jax.ShapeDtypeStruct((N, 1), jnp.float32)],
    )(softmaxes, lbl2)

    ece, ys = pl.pallas_call(
        _phase_b_kernel,
        in_specs=[pl.BlockSpec((S, S), lambda: (0, 0)),
                  pl.BlockSpec((S, S), lambda: (0, 0))],
        out_specs=[pl.BlockSpec((1, 1), lambda: (0, 0)),
                   pl.BlockSpec((1, NB), lambda: (0, 0))],
        out_shape=[jax.ShapeDtypeStruct((1, 1), jnp.float32),
                   jax.ShapeDtypeStruct((1, NB), jnp.float32)],
    )(conf.reshape(S, S), accv.reshape(S, S))
    return (ece.reshape(1), ys.reshape(NB))
